# baseline (speedup 1.0000x reference)
import functools

import jax
import jax.numpy as jnp
from jax import lax
from jax.experimental import pallas as pl
from jax.experimental.pallas import tpu as pltpu
from jax.experimental.pallas import tpu_sc as plsc

T0, T1, D = 512, 512, 64
NC, NS, L = 2, 16, 16
NW = NC * NS
C = 1024
G = 128
NG = C // G


def _sc_embed_body(xf_hbm, tab_hbm, out_hbm, xv, idx_v, rows, sem, per_w):
    wid = lax.axis_index("s") * NC + lax.axis_index("c")
    lane = lax.iota(jnp.int32, L)
    lane2 = lane * 2
    nblk = per_w // C

    def quantize(v):
        t = (v * 0.5 + 0.5) * 512.0
        t = jnp.minimum(t, 511.0)
        t = jnp.maximum(t, 0.0)
        return t.astype(jnp.int32)

    def block(b, carry):
        base = wid * per_w + b * C
        pltpu.sync_copy(xf_hbm.at[pl.ds(base * 2, C * 2)], xv)

        def comp(i, c2):
            off = i * (2 * L) + lane2
            x0 = plsc.load_gather(xv, [off])
            x1 = plsc.load_gather(xv, [off + 1])
            flat = quantize(x0) * 512 + quantize(x1)
            idx_v[pl.ds(i * L, L)] = flat
            return c2

        lax.fori_loop(0, C // L, comp, 0, unroll=2)

        cps = [
            pltpu.async_copy(
                tab_hbm.at[idx_v.at[pl.ds(j * G, G)]],
                rows.at[pl.ds(j * G, G)],
                sem,
            )
            for j in range(NG)
        ]
        for cp in cps:
            cp.wait()
        pltpu.sync_copy(rows, out_hbm.at[pl.ds(base, C)])
        return carry

    lax.fori_loop(0, nblk, block, 0)


@functools.partial(jax.jit, static_argnames=("n",))
def _sc_embed(xf, tab, n):
    per_w = n // NW
    mesh = plsc.VectorSubcoreMesh(core_axis_name="c", subcore_axis_name="s")
    body = functools.partial(_sc_embed_body, per_w=per_w)
    return pl.kernel(
        body,
        out_type=jax.ShapeDtypeStruct((n, D), jnp.float32),
        mesh=mesh,
        compiler_params=pltpu.CompilerParams(
            needs_layout_passes=False, use_tc_tiling_on_sc=False
        ),
        scratch_types=[
            pltpu.VMEM((2 * C,), jnp.float32),
            pltpu.VMEM((C,), jnp.int32),
            pltpu.VMEM((C, D), jnp.float32),
            pltpu.SemaphoreType.DMA,
        ],
    )(xf, tab)


def kernel(x, table):
    n = x.shape[0] * x.shape[1]
    xf = x.reshape(-1)
    tab = table.reshape(T0 * T1, D)
    out = _sc_embed(xf, tab, n)
    return out.reshape(x.shape[0], x.shape[1], D)
